# baseline (speedup 1.0000x reference)
import functools
import jax
import jax.numpy as jnp
from jax import lax
from jax.experimental import pallas as pl
from jax.experimental.pallas import tpu as pltpu
from jax.experimental.pallas import tpu_sc as plsc

_VOCAB = 100
_EMBED_DIM = 10
_BATCH = 16384
_HIST = 200
_NC, _NS, _L = 2, 16, 16
_NW = _NC * _NS
_ROWS_W = _BATCH // _NW
_R = 64
_K = _ROWS_W // _R
_VPAD = 112
_COLS = tuple(range(0, _HIST - 16, 16)) + (_HIST - 16,)

_mesh = plsc.VectorSubcoreMesh(core_axis_name="c", subcore_axis_name="s")


@functools.partial(
    pl.kernel,
    mesh=_mesh,
    out_type=jax.ShapeDtypeStruct((_BATCH, _HIST), jnp.float32),
    compiler_params=pltpu.CompilerParams(needs_layout_passes=False),
    scratch_types=[
        pltpu.VMEM((_VOCAB, _EMBED_DIM), jnp.float32),
        pltpu.VMEM((16,), jnp.float32),
        pltpu.VMEM((_VPAD,), jnp.float32),
        pltpu.VMEM((_R, _HIST), jnp.int32),
        pltpu.VMEM((_R, _HIST), jnp.int32),
        pltpu.VMEM((_R, _HIST), jnp.int32),
        pltpu.VMEM((_R, _HIST), jnp.float32),
        pltpu.VMEM((_R, _HIST), jnp.float32),
        pltpu.VMEM((_R, _HIST), jnp.float32),
        pltpu.SemaphoreType.DMA,
        pltpu.SemaphoreType.DMA,
        pltpu.SemaphoreType.DMA,
        pltpu.SemaphoreType.DMA,
        pltpu.SemaphoreType.DMA,
        pltpu.SemaphoreType.DMA,
    ],
)
def _lookup(x_hbm, tbl_hbm, par_hbm, out_hbm, tbl_v, par_v, s_v,
            idx_a, idx_b, idx_c, out_a, out_b, out_c,
            sin_a, sin_b, sin_c, sout_a, sout_b, sout_c):
    wid = lax.axis_index("s") * _NC + lax.axis_index("c")
    base = wid * _ROWS_W
    idx_bufs = (idx_a, idx_b, idx_c)
    out_bufs = (out_a, out_b, out_c)
    sin = (sin_a, sin_b, sin_c)
    sout = (sout_a, sout_b, sout_c)
    _NB = 3

    h_in = [None, None, None]
    h_out = [None, None, None]
    h_in[0] = pltpu.async_copy(x_hbm.at[pl.ds(base, _R)], idx_a, sin_a)
    h_in[1] = pltpu.async_copy(x_hbm.at[pl.ds(base + _R, _R)], idx_b, sin_b)

    pltpu.sync_copy(tbl_hbm, tbl_v)
    pltpu.sync_copy(par_hbm, par_v)

    lanes = lax.iota(jnp.int32, 16)
    for g in range(_VPAD // 16):
        v = jnp.minimum(lanes + g * 16, _VOCAB - 1)
        acc = plsc.load_gather(par_v, [jnp.full((16,), _EMBED_DIM + 1, jnp.int32)])
        for d in range(_EMBED_DIM):
            t = plsc.load_gather(tbl_v, [v, jnp.full((16,), d, jnp.int32)])
            w = plsc.load_gather(par_v, [jnp.full((16,), d + 1, jnp.int32)])
            acc = acc + t * w
        s_v[pl.ds(g * 16, 16)] = acc

    for k in range(_K):
        cur = k % _NB
        if k + 2 < _K:
            nxt = (k + 2) % _NB
            h_in[nxt] = pltpu.async_copy(
                x_hbm.at[pl.ds(base + (k + 2) * _R, _R)], idx_bufs[nxt], sin[nxt])
        h_in[cur].wait()
        if k >= _NB:
            h_out[cur].wait()
        idx_v, out_v = idx_bufs[cur], out_bufs[cur]

        @plsc.parallel_loop(0, _R, 1, unroll=8)
        def gather_body(r):
            for c in _COLS:
                iv = idx_v[r, pl.ds(c, 16)]
                out_v[r, pl.ds(c, 16)] = plsc.load_gather(s_v, [iv])

        h_out[cur] = pltpu.async_copy(
            out_v, out_hbm.at[pl.ds(base + k * _R, _R)], sout[cur])

    for t in range(_NB):
        if _K - 1 - t >= 0:
            h_out[(_K - 1 - t) % _NB].wait()


def kernel(x, embed_table, lin_w, lin_b):
    params = jnp.concatenate(
        [jnp.zeros((1,), jnp.float32), lin_w[0], lin_b,
         jnp.zeros((16 - _EMBED_DIM - 2,), jnp.float32)]
    )
    return _lookup(x, embed_table, params)[:, :, None]


# speedup vs baseline: 1.0553x; 1.0553x over previous
import functools
import jax
import jax.numpy as jnp
from jax import lax
from jax.experimental import pallas as pl
from jax.experimental.pallas import tpu as pltpu
from jax.experimental.pallas import tpu_sc as plsc

_VOCAB = 100
_EMBED_DIM = 10
_BATCH = 16384
_HIST = 200
_NC, _NS, _L = 2, 16, 16
_NW = _NC * _NS
_ROWS_W = _BATCH // _NW
_R = 64
_K = _ROWS_W // _R
_VPAD = 112
_COLS = tuple(range(0, _HIST - 16, 16)) + (_HIST - 16,)

_mesh = plsc.VectorSubcoreMesh(core_axis_name="c", subcore_axis_name="s")


@functools.partial(
    pl.kernel,
    mesh=_mesh,
    out_type=jax.ShapeDtypeStruct((_BATCH, _HIST), jnp.float32),
    compiler_params=pltpu.CompilerParams(needs_layout_passes=False),
    scratch_types=[
        pltpu.VMEM((_VOCAB, _EMBED_DIM), jnp.float32),
        pltpu.VMEM((16,), jnp.float32),
        pltpu.VMEM((_VPAD,), jnp.float32),
        pltpu.VMEM((_R, _HIST), jnp.int32),
        pltpu.VMEM((_R, _HIST), jnp.int32),
        pltpu.VMEM((_R, _HIST), jnp.int32),
        pltpu.VMEM((_R, _HIST), jnp.float32),
        pltpu.VMEM((_R, _HIST), jnp.float32),
        pltpu.VMEM((_R, _HIST), jnp.float32),
        pltpu.SemaphoreType.DMA,
        pltpu.SemaphoreType.DMA,
        pltpu.SemaphoreType.DMA,
        pltpu.SemaphoreType.DMA,
        pltpu.SemaphoreType.DMA,
        pltpu.SemaphoreType.DMA,
    ],
)
def _lookup(x_hbm, tbl_hbm, par_hbm, out_hbm, tbl_v, par_v, s_v,
            idx_a, idx_b, idx_c, out_a, out_b, out_c,
            sin_a, sin_b, sin_c, sout_a, sout_b, sout_c):
    wid = lax.axis_index("s") * _NC + lax.axis_index("c")
    base = wid * _ROWS_W
    idx_bufs = (idx_a, idx_b, idx_c)
    out_bufs = (out_a, out_b, out_c)
    sin = (sin_a, sin_b, sin_c)
    sout = (sout_a, sout_b, sout_c)
    _NB = 3

    h_in = [None, None, None]
    h_out = [None, None, None]
    h_in[0] = pltpu.async_copy(x_hbm.at[pl.ds(base, _R)], idx_a, sin_a)
    h_in[1] = pltpu.async_copy(x_hbm.at[pl.ds(base + _R, _R)], idx_b, sin_b)

    pltpu.sync_copy(tbl_hbm, tbl_v)
    pltpu.sync_copy(par_hbm, par_v)

    lanes = lax.iota(jnp.int32, 16)
    for g in range(_VPAD // 16):
        v = jnp.minimum(lanes + g * 16, _VOCAB - 1)
        acc = plsc.load_gather(par_v, [jnp.full((16,), _EMBED_DIM + 1, jnp.int32)])
        for d in range(_EMBED_DIM):
            t = plsc.load_gather(tbl_v, [v, jnp.full((16,), d, jnp.int32)])
            w = plsc.load_gather(par_v, [jnp.full((16,), d + 1, jnp.int32)])
            acc = acc + t * w
        s_v[pl.ds(g * 16, 16)] = acc

    for k in range(_K):
        cur = k % _NB
        if k + 2 < _K:
            nxt = (k + 2) % _NB
            h_in[nxt] = pltpu.async_copy(
                x_hbm.at[pl.ds(base + (k + 2) * _R, _R)], idx_bufs[nxt], sin[nxt])
        h_in[cur].wait()
        if k >= _NB:
            h_out[cur].wait()
        idx_v, out_v = idx_bufs[cur], out_bufs[cur]

        @plsc.parallel_loop(0, _R, 1, unroll=2)
        def gather_body(r):
            for c in _COLS:
                iv = idx_v[r, pl.ds(c, 16)]
                out_v[r, pl.ds(c, 16)] = plsc.load_gather(s_v, [iv])

        h_out[cur] = pltpu.async_copy(
            out_v, out_hbm.at[pl.ds(base + k * _R, _R)], sout[cur])

    for t in range(_NB):
        if _K - 1 - t >= 0:
            h_out[(_K - 1 - t) % _NB].wait()


def kernel(x, embed_table, lin_w, lin_b):
    params = jnp.concatenate(
        [jnp.zeros((1,), jnp.float32), lin_w[0], lin_b,
         jnp.zeros((16 - _EMBED_DIM - 2,), jnp.float32)]
    )
    return _lookup(x, embed_table, params)[:, :, None]
